# Initial kernel scaffold; baseline (speedup 1.0000x reference)
import jax
import jax.numpy as jnp
from jax.experimental import pallas as pl


def kernel(x, edge_index, W1, b1, W2, b2):
    raise NotImplementedError("write your pallas kernel here")


# speedup vs baseline: 8.7324x; 8.7324x over previous
import functools

import jax
import jax.numpy as jnp
from jax import lax
from jax.experimental import pallas as pl
from jax.experimental.pallas import tpu as pltpu
from jax.experimental.pallas import tpu_sc as plsc

N_NODES = 10000
N_EDGES = 320000
D = 128
NC = 2
NS = 16
HALF = N_NODES // NC
TRASH = 64
ACC_R = HALF + TRASH
CH = 125
EPT = N_EDGES // NS
NCHUNK = EPT // CH
WB = 5
WBR = HALF // WB

_sc_mesh = plsc.VectorSubcoreMesh(core_axis_name="c", subcore_axis_name="s")


def _hist_body(hidx, ones, zeros, out, h0_v, h1_v, ones_v, acc_sh):
    cid = lax.axis_index("c")
    sid = lax.axis_index("s")
    pltpu.sync_copy(hidx.at[0, cid, sid], h0_v)
    pltpu.sync_copy(hidx.at[1, cid, sid], h1_v)
    pltpu.sync_copy(ones, ones_v)

    for j, h_v in ((0, h0_v), (1, h1_v)):
        @pl.when(sid == 0)
        def _():
            pltpu.sync_copy(zeros, acc_sh)

        plsc.subcore_barrier()

        def body(c, carry, h_v=h_v):
            pltpu.sync_copy(ones_v, acc_sh.at[h_v.at[c]], add=True)
            return carry

        lax.fori_loop(0, NCHUNK, body, 0)
        plsc.subcore_barrier()

        @pl.when(sid < WB)
        def _():
            pltpu.sync_copy(
                acc_sh.at[pl.ds(sid * WBR, WBR)],
                out.at[j, pl.ds(cid * HALF + sid * WBR, WBR)],
            )

        plsc.subcore_barrier()


_hist = functools.partial(
    pl.kernel,
    out_type=jax.ShapeDtypeStruct((2, N_NODES, D), jnp.float32),
    mesh=_sc_mesh,
    scratch_types=[
        pltpu.VMEM((NCHUNK, CH), jnp.int32),
        pltpu.VMEM((NCHUNK, CH), jnp.int32),
        pltpu.VMEM((CH, D), jnp.float32),
        pltpu.VMEM_SHARED((ACC_R, D), jnp.float32),
    ],
)(_hist_body)


def _agg_body(z, gidx, sidx, zeros, out, g_v, s_v, buf0, buf1, sem0, sem1,
              acc_sh):
    cid = lax.axis_index("c")
    sid = lax.axis_index("s")
    pltpu.sync_copy(gidx.at[sid], g_v)
    pltpu.sync_copy(sidx.at[cid, sid], s_v)

    @pl.when(sid == 0)
    def _():
        pltpu.sync_copy(zeros, acc_sh)

    plsc.subcore_barrier()

    pltpu.async_copy(z.at[g_v.at[0]], buf0, sem0)

    def body(c0, carry):
        pltpu.async_copy(z.at[g_v.at[c0 + 1]], buf1, sem1)
        pltpu.make_async_copy(z.at[g_v.at[c0]], buf0, sem0).wait()
        pltpu.sync_copy(buf0, acc_sh.at[s_v.at[c0]], add=True)

        @pl.when(c0 + 2 < NCHUNK)
        def _():
            pltpu.async_copy(z.at[g_v.at[c0 + 2]], buf0, sem0)

        pltpu.make_async_copy(z.at[g_v.at[c0 + 1]], buf1, sem1).wait()
        pltpu.sync_copy(buf1, acc_sh.at[s_v.at[c0 + 1]], add=True)
        return carry

    lax.fori_loop(0, NCHUNK // 2, lambda i, c: body(i * 2, c), 0)
    plsc.subcore_barrier()

    @pl.when(sid < WB)
    def _():
        pltpu.sync_copy(
            acc_sh.at[pl.ds(sid * WBR, WBR)],
            out.at[pl.ds(cid * HALF + sid * WBR, WBR)],
        )


_agg = functools.partial(
    pl.kernel,
    out_type=jax.ShapeDtypeStruct((N_NODES, D), jnp.float32),
    mesh=_sc_mesh,
    scratch_types=[
        pltpu.VMEM((NCHUNK, CH), jnp.int32),
        pltpu.VMEM((NCHUNK, CH), jnp.int32),
        pltpu.VMEM((CH, D), jnp.float32),
        pltpu.VMEM((CH, D), jnp.float32),
        pltpu.SemaphoreType.DMA,
        pltpu.SemaphoreType.DMA,
        pltpu.VMEM_SHARED((ACC_R, D), jnp.float32),
    ],
)(_agg_body)


def _norm(hist_blk):
    deg = hist_blk[:, :1]
    return jnp.where(deg > 0, lax.rsqrt(jnp.maximum(deg, 1.0)), 0.0)


_RB = 1000


def _mm1_body(x_ref, dego_ref, w_ref, z_ref):
    z_ref[...] = jnp.dot(x_ref[...] * _norm(dego_ref[...]), w_ref[...],
                         preferred_element_type=jnp.float32)


_mm1 = pl.pallas_call(
    _mm1_body,
    grid=(N_NODES // _RB,),
    in_specs=[
        pl.BlockSpec((_RB, D), lambda i: (i, 0)),
        pl.BlockSpec((_RB, D), lambda i: (i, 0)),
        pl.BlockSpec((D, D), lambda i: (0, 0)),
    ],
    out_specs=pl.BlockSpec((_RB, D), lambda i: (i, 0)),
    out_shape=jax.ShapeDtypeStruct((N_NODES, D), jnp.float32),
)


def _mm2_body(agg_ref, degi_ref, dego_ref, w_ref, b_ref, z_ref):
    h = jnp.maximum(agg_ref[...] * _norm(degi_ref[...]) + b_ref[...], 0.0)
    z_ref[...] = jnp.dot(h * _norm(dego_ref[...]), w_ref[...],
                         preferred_element_type=jnp.float32)


_mm2 = pl.pallas_call(
    _mm2_body,
    grid=(N_NODES // _RB,),
    in_specs=[
        pl.BlockSpec((_RB, D), lambda i: (i, 0)),
        pl.BlockSpec((_RB, D), lambda i: (i, 0)),
        pl.BlockSpec((_RB, D), lambda i: (i, 0)),
        pl.BlockSpec((D, D), lambda i: (0, 0)),
        pl.BlockSpec((1, D), lambda i: (0, 0)),
    ],
    out_specs=pl.BlockSpec((_RB, D), lambda i: (i, 0)),
    out_shape=jax.ShapeDtypeStruct((N_NODES, D), jnp.float32),
)


def _out_body(agg_ref, degi_ref, b_ref, o_ref):
    o_ref[...] = jnp.maximum(agg_ref[...] * _norm(degi_ref[...]) + b_ref[...],
                             0.0)


_outk = pl.pallas_call(
    _out_body,
    grid=(N_NODES // _RB,),
    in_specs=[
        pl.BlockSpec((_RB, D), lambda i: (i, 0)),
        pl.BlockSpec((_RB, D), lambda i: (i, 0)),
        pl.BlockSpec((1, D), lambda i: (0, 0)),
    ],
    out_specs=pl.BlockSpec((_RB, D), lambda i: (i, 0)),
    out_shape=jax.ShapeDtypeStruct((N_NODES, D), jnp.float32),
)


def kernel(x, edge_index, W1, b1, W2, b2):
    e = edge_index.astype(jnp.int32)
    src, dst = e[0], e[1]
    rot = jnp.arange(N_EDGES, dtype=jnp.int32) % TRASH + HALF

    def clamp_halves(idx):
        lo = jnp.where(idx < HALF, idx, rot)
        hi = jnp.where(idx >= HALF, idx - HALF, rot)
        return jnp.stack([lo, hi]).reshape(NC, NS, NCHUNK, CH)

    sidx = clamp_halves(dst)
    hidx = jnp.stack([clamp_halves(src), sidx])
    gidx = src.reshape(NS, NCHUNK, CH)
    ones = jnp.ones((CH, D), jnp.float32)
    zeros = jnp.zeros((ACC_R, D), jnp.float32)

    deg = _hist(hidx, ones, zeros)
    dego, degi = deg[0], deg[1]

    z1 = _mm1(x, dego, W1)
    agg1 = _agg(z1, gidx, sidx, zeros)
    z2 = _mm2(agg1, degi, dego, W2, b1.reshape(1, D))
    agg2 = _agg(z2, gidx, sidx, zeros)
    return _outk(agg2, degi, b2.reshape(1, D))
